# speedup vs baseline: 1.5584x; 1.5584x over previous
import functools

import jax
import jax.numpy as jnp
from jax import lax
from jax.experimental import pallas as pl
from jax.experimental.pallas import tpu as pltpu
from jax.experimental.pallas import tpu_sc as plsc

NC = 2
NS = 16
NW = NC * NS


@functools.partial(jax.jit, static_argnames=("n", "d"))
def _gather_rows(ids_flat, table, n, d):
    rows_per_w = n // NW
    chunk = 32
    n_chunks = rows_per_w // chunk

    mesh = plsc.VectorSubcoreMesh(core_axis_name="c", subcore_axis_name="s")

    @functools.partial(
        pl.kernel,
        out_type=jax.ShapeDtypeStruct((n, d), jnp.float32),
        mesh=mesh,
        scratch_types=[
            pltpu.VMEM((rows_per_w,), jnp.int32),
            pltpu.VMEM((chunk, d), jnp.float32),
            pltpu.SemaphoreType.DMA,
        ],
    )
    def k(ids_hbm, table_hbm, out_hbm, idx_v, rows_v, sem):
        wid = lax.axis_index("s") * NC + lax.axis_index("c")
        base = wid * rows_per_w
        pltpu.sync_copy(ids_hbm.at[pl.ds(base, rows_per_w)], idx_v)
        for g in range(n_chunks):
            pltpu.async_copy(
                table_hbm.at[idx_v.at[pl.ds(g * chunk, chunk)]], rows_v, sem
            ).wait()
            pltpu.sync_copy(rows_v, out_hbm.at[pl.ds(base + g * chunk, chunk)])

    return k(ids_flat, table)


def kernel(input_ids, table):
    b, s = input_ids.shape
    v, d = table.shape
    ids_flat = input_ids.reshape(b * s).astype(jnp.int32)
    out = _gather_rows(ids_flat, table, b * s, d)
    return out.reshape(b, s, d)


# speedup vs baseline: 1.5853x; 1.0173x over previous
import functools

import jax
import jax.numpy as jnp
from jax import lax
from jax.experimental import pallas as pl
from jax.experimental.pallas import tpu as pltpu
from jax.experimental.pallas import tpu_sc as plsc

NC = 2
NS = 16
NW = NC * NS


@functools.partial(jax.jit, static_argnames=("n", "d"))
def _gather_rows(ids_flat, table, n, d):
    rows_per_w = n // NW
    chunk = 16
    n_chunks = rows_per_w // chunk

    mesh = plsc.VectorSubcoreMesh(core_axis_name="c", subcore_axis_name="s")

    @functools.partial(
        pl.kernel,
        out_type=jax.ShapeDtypeStruct((n, d), jnp.float32),
        mesh=mesh,
        scratch_types=[
            pltpu.VMEM((rows_per_w,), jnp.int32),
            pltpu.VMEM((chunk, d), jnp.float32),
            pltpu.VMEM((chunk, d), jnp.float32),
            pltpu.SemaphoreType.DMA,
            pltpu.SemaphoreType.DMA,
            pltpu.SemaphoreType.DMA,
            pltpu.SemaphoreType.DMA,
        ],
    )
    def k(ids_hbm, table_hbm, out_hbm, idx_v, b0, b1, g0, g1, s0, s1):
        wid = lax.axis_index("s") * NC + lax.axis_index("c")
        base = wid * rows_per_w
        pltpu.sync_copy(ids_hbm.at[pl.ds(base, rows_per_w)], idx_v)
        bufs = (b0, b1)
        gsems = (g0, g1)
        ssems = (s0, s1)
        gathers = {}
        stores = {}
        gathers[0] = pltpu.async_copy(
            table_hbm.at[idx_v.at[pl.ds(0, chunk)]], bufs[0], gsems[0]
        )
        for g in range(n_chunks):
            p = g % 2
            gathers[g].wait()
            stores[g] = pltpu.async_copy(
                bufs[p], out_hbm.at[pl.ds(base + g * chunk, chunk)], ssems[p]
            )
            if g + 1 < n_chunks:
                if g >= 1:
                    stores[g - 1].wait()
                gathers[g + 1] = pltpu.async_copy(
                    table_hbm.at[idx_v.at[pl.ds((g + 1) * chunk, chunk)]],
                    bufs[(g + 1) % 2],
                    gsems[(g + 1) % 2],
                )
        if n_chunks >= 2:
            stores[n_chunks - 2].wait()
        stores[n_chunks - 1].wait()

    return k(ids_flat, table)


def kernel(input_ids, table):
    b, s = input_ids.shape
    v, d = table.shape
    ids_flat = input_ids.reshape(b * s).astype(jnp.int32)
    out = _gather_rows(ids_flat, table, b * s, d)
    return out.reshape(b, s, d)


# speedup vs baseline: 1.6270x; 1.0263x over previous
import functools

import jax
import jax.numpy as jnp
from jax import lax
from jax.experimental import pallas as pl
from jax.experimental.pallas import tpu as pltpu
from jax.experimental.pallas import tpu_sc as plsc

NC = 2
NS = 16
NW = NC * NS


@functools.partial(jax.jit, static_argnames=("n", "d"))
def _gather_rows(ids_flat, table, n, d):
    rows_per_w = n // NW
    chunk = 16
    nbuf = 3
    n_chunks = rows_per_w // chunk

    mesh = plsc.VectorSubcoreMesh(core_axis_name="c", subcore_axis_name="s")

    @functools.partial(
        pl.kernel,
        out_type=jax.ShapeDtypeStruct((n, d), jnp.float32),
        mesh=mesh,
        scratch_types=[
            pltpu.VMEM((rows_per_w,), jnp.int32),
            *[pltpu.VMEM((chunk, d), jnp.float32) for _ in range(nbuf)],
            *[pltpu.SemaphoreType.DMA for _ in range(2 * nbuf)],
        ],
    )
    def k(ids_hbm, table_hbm, out_hbm, idx_v, *scr):
        bufs = scr[:nbuf]
        gsems = scr[nbuf : 2 * nbuf]
        ssems = scr[2 * nbuf :]
        wid = lax.axis_index("s") * NC + lax.axis_index("c")
        base = wid * rows_per_w
        pltpu.sync_copy(ids_hbm.at[pl.ds(base, rows_per_w)], idx_v)

        def fire_gather(g):
            p = g % nbuf
            return pltpu.async_copy(
                table_hbm.at[idx_v.at[pl.ds(g * chunk, chunk)]], bufs[p], gsems[p]
            )

        gathers = {}
        stores = {}
        for g in range(min(nbuf - 1, n_chunks)):
            gathers[g] = fire_gather(g)
        for g in range(n_chunks):
            p = g % nbuf
            gathers[g].wait()
            stores[g] = pltpu.async_copy(
                bufs[p], out_hbm.at[pl.ds(base + g * chunk, chunk)], ssems[p]
            )
            nxt = g + nbuf - 1
            if nxt < n_chunks:
                if g >= 1:
                    stores[g - 1].wait()
                gathers[nxt] = fire_gather(nxt)
        for g in range(max(0, n_chunks - nbuf), n_chunks):
            stores[g].wait()

    return k(ids_flat, table)


def kernel(input_ids, table):
    b, s = input_ids.shape
    v, d = table.shape
    ids_flat = input_ids.reshape(b * s).astype(jnp.int32)
    out = _gather_rows(ids_flat, table, b * s, d)
    return out.reshape(b, s, d)
